# Initial kernel scaffold; baseline (speedup 1.0000x reference)
import jax
import jax.numpy as jnp
from jax.experimental import pallas as pl


def kernel(x, edge_index, batch, lin0_W, lin0_b, conv_W, conv_b, lstm_Wih, lstm_Whh, lstm_bih, lstm_bhh, lin1_W, lin1_b, lin2_W, lin2_b):
    raise NotImplementedError("write your pallas kernel here")


# speedup vs baseline: 1.0561x; 1.0561x over previous
import functools

import jax
import jax.numpy as jnp
from jax.experimental import pallas as pl

N_BLK = 2000
B = 256
D = 64


def _dg0(a, b):
    return jax.lax.dot_general(
        a, b, dimension_numbers=(((0,), (0,)), ((), ())),
        preferred_element_type=jnp.float32)


def _dot(a, b):
    return jnp.dot(a, b, preferred_element_type=jnp.float32)


def _lin0_body(x_ref, w_ref, b_ref, cw_ref, out_ref, h_ref):
    o = jax.nn.relu(_dot(x_ref[...], w_ref[...]) + b_ref[...])
    out_ref[...] = o
    h_ref[...] = _dot(o, cw_ref[...])


def _prop_body(agg_ref, cb_ref, cw_ref, out_ref, h_ref):
    o = jax.nn.relu(agg_ref[...] + cb_ref[...])
    out_ref[...] = o
    h_ref[...] = _dot(o, cw_ref[...])


def _onehot(b_ref):
    ids = jax.lax.broadcasted_iota(jnp.int32, (1, B), 1)
    return b_ref[...] == ids


def _emax_body(out_ref, b_ref, q_ref, emax_ref):
    i = pl.program_id(0)
    oh = _onehot(b_ref)
    ohf = oh.astype(jnp.float32)
    qn = _dot(ohf, q_ref[...])
    e = jnp.sum(out_ref[...] * qn, axis=1, keepdims=True)
    masked = jnp.where(oh, e, -1e30)
    bm = jnp.max(masked, axis=0, keepdims=True)

    @pl.when(i == 0)
    def _():
        emax_ref[...] = jnp.full((1, B), -1e30, jnp.float32)

    emax_ref[...] = jnp.maximum(emax_ref[...], bm)


def _att_body(out_ref, b_ref, q_ref, emax_ref, denom_ref, rnum_ref):
    i = pl.program_id(0)
    oh = _onehot(b_ref)
    ohf = oh.astype(jnp.float32)
    qn = _dot(ohf, q_ref[...])
    e = jnp.sum(out_ref[...] * qn, axis=1, keepdims=True)
    emn = jnp.sum(ohf * emax_ref[...], axis=1, keepdims=True)
    ee = jnp.exp(e - emn)

    @pl.when(i == 0)
    def _():
        denom_ref[...] = jnp.zeros((B, 1), jnp.float32)
        rnum_ref[...] = jnp.zeros((B, D), jnp.float32)

    denom_ref[...] += _dg0(ohf, ee)
    rnum_ref[...] += _dg0(ohf, ee * out_ref[...])


def _lstm_body(qp_ref, rn_ref, dn_ref, hs_ref, cs_ref,
               wia_ref, wib_ref, whh_ref, bs_ref, hs_o, cs_o):
    r = rn_ref[...] * (1.0 / jnp.maximum(dn_ref[...], 1e-30))
    gates = (_dot(qp_ref[...], wia_ref[...]) + _dot(r, wib_ref[...])
             + _dot(hs_ref[...], whh_ref[...]) + bs_ref[...])
    ig = jax.nn.sigmoid(gates[:, 0:D])
    fg = jax.nn.sigmoid(gates[:, D:2 * D])
    gg = jnp.tanh(gates[:, 2 * D:3 * D])
    og = jax.nn.sigmoid(gates[:, 3 * D:4 * D])
    c = fg * cs_ref[...] + ig * gg
    cs_o[...] = c
    hs_o[...] = og * jnp.tanh(c)


def _head_body(q_ref, rn_ref, dn_ref, w1a_ref, w1b_ref, b1_ref,
               w2_ref, b2_ref, y_ref):
    r = rn_ref[...] * (1.0 / jnp.maximum(dn_ref[...], 1e-30))
    t = jax.nn.relu(_dot(q_ref[...], w1a_ref[...])
                    + _dot(r, w1b_ref[...]) + b1_ref[...])
    y_ref[...] = _dot(t, w2_ref[...]) + b2_ref[...]


def _node_spec(n_cols):
    return pl.BlockSpec((N_BLK, n_cols), lambda i: (i, 0))


def _full_spec(shape):
    return pl.BlockSpec(shape, lambda i: (0, 0))


@functools.partial(jax.jit)
def _impl(x, edge_index, batch, lin0_W, lin0_b, conv_W, conv_b,
          lstm_Wih, lstm_Whh, lstm_bih, lstm_bhh, lin1_W, lin1_b,
          lin2_W, lin2_b):
    N = x.shape[0]
    d_in = x.shape[1]
    grid = (N // N_BLK,)
    f32 = jnp.float32

    loop = jnp.arange(N, dtype=edge_index.dtype)
    src = jnp.concatenate([edge_index[0], loop])
    dst = jnp.concatenate([edge_index[1], loop])
    deg = jax.ops.segment_sum(jnp.ones_like(dst, dtype=f32), dst,
                              num_segments=N)
    dinv = 1.0 / jnp.sqrt(deg)
    norm = (dinv[src] * dinv[dst])[:, None]

    out, h = pl.pallas_call(
        _lin0_body,
        grid=grid,
        in_specs=[_node_spec(d_in), _full_spec((d_in, D)),
                  _full_spec((1, D)), _full_spec((D, D))],
        out_specs=(_node_spec(D), _node_spec(D)),
        out_shape=(jax.ShapeDtypeStruct((N, D), f32),
                   jax.ShapeDtypeStruct((N, D), f32)),
    )(x, lin0_W, lin0_b.reshape(1, D), conv_W)

    prop = pl.pallas_call(
        _prop_body,
        grid=grid,
        in_specs=[_node_spec(D), _full_spec((1, D)), _full_spec((D, D))],
        out_specs=(_node_spec(D), _node_spec(D)),
        out_shape=(jax.ShapeDtypeStruct((N, D), f32),
                   jax.ShapeDtypeStruct((N, D), f32)),
    )

    for _ in range(6):
        msg = norm * jnp.take(h, src, axis=0)
        agg = jax.ops.segment_sum(msg, dst, num_segments=N)
        out, h = prop(agg, conv_b.reshape(1, D), conv_W)

    batch2 = batch.reshape(N, 1)
    wia = lstm_Wih[:, :D].T
    wib = lstm_Wih[:, D:].T
    whh = lstm_Whh.T
    bsum = (lstm_bih + lstm_bhh).reshape(1, 4 * D)

    lstm = pl.pallas_call(
        _lstm_body,
        in_specs=[pl.BlockSpec((B, D), lambda: (0, 0)),
                  pl.BlockSpec((B, D), lambda: (0, 0)),
                  pl.BlockSpec((B, 1), lambda: (0, 0)),
                  pl.BlockSpec((B, D), lambda: (0, 0)),
                  pl.BlockSpec((B, D), lambda: (0, 0)),
                  pl.BlockSpec((D, 4 * D), lambda: (0, 0)),
                  pl.BlockSpec((D, 4 * D), lambda: (0, 0)),
                  pl.BlockSpec((D, 4 * D), lambda: (0, 0)),
                  pl.BlockSpec((1, 4 * D), lambda: (0, 0))],
        out_specs=(pl.BlockSpec((B, D), lambda: (0, 0)),
                   pl.BlockSpec((B, D), lambda: (0, 0))),
        out_shape=(jax.ShapeDtypeStruct((B, D), f32),
                   jax.ShapeDtypeStruct((B, D), f32)),
    )

    emax_call = pl.pallas_call(
        _emax_body,
        grid=grid,
        in_specs=[_node_spec(D), _node_spec(1), _full_spec((B, D))],
        out_specs=_full_spec((1, B)),
        out_shape=jax.ShapeDtypeStruct((1, B), f32),
    )

    att_call = pl.pallas_call(
        _att_body,
        grid=grid,
        in_specs=[_node_spec(D), _node_spec(1), _full_spec((B, D)),
                  _full_spec((1, B))],
        out_specs=(_full_spec((B, 1)), _full_spec((B, D))),
        out_shape=(jax.ShapeDtypeStruct((B, 1), f32),
                   jax.ShapeDtypeStruct((B, D), f32)),
    )

    q = jnp.zeros((B, D), f32)
    hs = jnp.zeros((B, D), f32)
    cs = jnp.zeros((B, D), f32)
    rnum = jnp.zeros((B, D), f32)
    denom = jnp.ones((B, 1), f32)

    for _ in range(6):
        hs, cs = lstm(q, rnum, denom, hs, cs, wia, wib, whh, bsum)
        q = hs
        emax = emax_call(out, batch2, q)
        denom, rnum = att_call(out, batch2, q, emax)

    y = pl.pallas_call(
        _head_body,
        in_specs=[pl.BlockSpec((B, D), lambda: (0, 0)),
                  pl.BlockSpec((B, D), lambda: (0, 0)),
                  pl.BlockSpec((B, 1), lambda: (0, 0)),
                  pl.BlockSpec((D, D), lambda: (0, 0)),
                  pl.BlockSpec((D, D), lambda: (0, 0)),
                  pl.BlockSpec((1, D), lambda: (0, 0)),
                  pl.BlockSpec((D, 12), lambda: (0, 0)),
                  pl.BlockSpec((1, 12), lambda: (0, 0))],
        out_specs=pl.BlockSpec((B, 12), lambda: (0, 0)),
        out_shape=jax.ShapeDtypeStruct((B, 12), f32),
    )(q, rnum, denom, lin1_W[:D], lin1_W[D:], lin1_b.reshape(1, D),
      lin2_W, lin2_b.reshape(1, 12))

    return y


def kernel(x, edge_index, batch, lin0_W, lin0_b, conv_W, conv_b,
           lstm_Wih, lstm_Whh, lstm_bih, lstm_bhh, lin1_W, lin1_b,
           lin2_W, lin2_b):
    return _impl(x, edge_index, batch, lin0_W, lin0_b, conv_W, conv_b,
                 lstm_Wih, lstm_Whh, lstm_bih, lstm_bhh, lin1_W, lin1_b,
                 lin2_W, lin2_b)


# speedup vs baseline: 1.6646x; 1.5762x over previous
import functools

import jax
import jax.numpy as jnp
from jax.experimental import pallas as pl

N_BLK = 2000
B = 256
D = 64


def _dg0(a, b):
    return jax.lax.dot_general(
        a, b, dimension_numbers=(((0,), (0,)), ((), ())),
        preferred_element_type=jnp.float32)


def _dot(a, b):
    return jnp.dot(a, b, preferred_element_type=jnp.float32)


def _lin0_body(x_ref, w_ref, b_ref, cw_ref, dv_ref, out_ref, h_ref):
    o = jax.nn.relu(_dot(x_ref[...], w_ref[...]) + b_ref[...])
    out_ref[...] = o
    h_ref[...] = _dot(o, cw_ref[...]) * dv_ref[...]


def _prop_body(agg_ref, cb_ref, cw_ref, dv_ref, out_ref, h_ref):
    o = jax.nn.relu(agg_ref[...] * dv_ref[...] + cb_ref[...])
    out_ref[...] = o
    h_ref[...] = _dot(o, cw_ref[...]) * dv_ref[...]


def _onehot(b_ref):
    ids = jax.lax.broadcasted_iota(jnp.int32, (1, B), 1)
    return b_ref[...] == ids


def _emax_body(out_ref, b_ref, q_ref, emax_ref):
    i = pl.program_id(0)
    oh = _onehot(b_ref)
    ohf = oh.astype(jnp.float32)
    qn = _dot(ohf, q_ref[...])
    e = jnp.sum(out_ref[...] * qn, axis=1, keepdims=True)
    masked = jnp.where(oh, e, -1e30)
    bm = jnp.max(masked, axis=0, keepdims=True)

    @pl.when(i == 0)
    def _():
        emax_ref[...] = jnp.full((1, B), -1e30, jnp.float32)

    emax_ref[...] = jnp.maximum(emax_ref[...], bm)


def _att_body(out_ref, b_ref, q_ref, emax_ref, denom_ref, rnum_ref):
    i = pl.program_id(0)
    oh = _onehot(b_ref)
    ohf = oh.astype(jnp.float32)
    qn = _dot(ohf, q_ref[...])
    e = jnp.sum(out_ref[...] * qn, axis=1, keepdims=True)
    emn = jnp.sum(ohf * emax_ref[...], axis=1, keepdims=True)
    ee = jnp.exp(e - emn)

    @pl.when(i == 0)
    def _():
        denom_ref[...] = jnp.zeros((B, 1), jnp.float32)
        rnum_ref[...] = jnp.zeros((B, D), jnp.float32)

    denom_ref[...] += _dg0(ohf, ee)
    rnum_ref[...] += _dg0(ohf, ee * out_ref[...])


def _lstm_body(qp_ref, rn_ref, dn_ref, hs_ref, cs_ref,
               wia_ref, wib_ref, whh_ref, bs_ref, hs_o, cs_o):
    r = rn_ref[...] * (1.0 / jnp.maximum(dn_ref[...], 1e-30))
    gates = (_dot(qp_ref[...], wia_ref[...]) + _dot(r, wib_ref[...])
             + _dot(hs_ref[...], whh_ref[...]) + bs_ref[...])
    ig = jax.nn.sigmoid(gates[:, 0:D])
    fg = jax.nn.sigmoid(gates[:, D:2 * D])
    gg = jnp.tanh(gates[:, 2 * D:3 * D])
    og = jax.nn.sigmoid(gates[:, 3 * D:4 * D])
    c = fg * cs_ref[...] + ig * gg
    cs_o[...] = c
    hs_o[...] = og * jnp.tanh(c)


def _head_body(q_ref, rn_ref, dn_ref, w1a_ref, w1b_ref, b1_ref,
               w2_ref, b2_ref, y_ref):
    r = rn_ref[...] * (1.0 / jnp.maximum(dn_ref[...], 1e-30))
    t = jax.nn.relu(_dot(q_ref[...], w1a_ref[...])
                    + _dot(r, w1b_ref[...]) + b1_ref[...])
    y_ref[...] = _dot(t, w2_ref[...]) + b2_ref[...]


def _node_spec(n_cols):
    return pl.BlockSpec((N_BLK, n_cols), lambda i: (i, 0))


def _full_spec(shape):
    return pl.BlockSpec(shape, lambda i: (0, 0))


@functools.partial(jax.jit)
def _impl(x, edge_index, batch, lin0_W, lin0_b, conv_W, conv_b,
          lstm_Wih, lstm_Whh, lstm_bih, lstm_bhh, lin1_W, lin1_b,
          lin2_W, lin2_b):
    N = x.shape[0]
    d_in = x.shape[1]
    grid = (N // N_BLK,)
    f32 = jnp.float32

    loop = jnp.arange(N, dtype=edge_index.dtype)
    src = jnp.concatenate([edge_index[0], loop])
    dst = jnp.concatenate([edge_index[1], loop])
    deg = jax.ops.segment_sum(jnp.ones_like(dst, dtype=f32), dst,
                              num_segments=N)
    dinv = 1.0 / jnp.sqrt(deg)
    dinv2 = dinv.reshape(N, 1)

    out, h = pl.pallas_call(
        _lin0_body,
        grid=grid,
        in_specs=[_node_spec(d_in), _full_spec((d_in, D)),
                  _full_spec((1, D)), _full_spec((D, D)), _node_spec(1)],
        out_specs=(_node_spec(D), _node_spec(D)),
        out_shape=(jax.ShapeDtypeStruct((N, D), f32),
                   jax.ShapeDtypeStruct((N, D), f32)),
    )(x, lin0_W, lin0_b.reshape(1, D), conv_W, dinv2)

    prop = pl.pallas_call(
        _prop_body,
        grid=grid,
        in_specs=[_node_spec(D), _full_spec((1, D)), _full_spec((D, D)),
                  _node_spec(1)],
        out_specs=(_node_spec(D), _node_spec(D)),
        out_shape=(jax.ShapeDtypeStruct((N, D), f32),
                   jax.ShapeDtypeStruct((N, D), f32)),
    )

    for _ in range(6):
        msg = jnp.take(h, src, axis=0)
        agg = jax.ops.segment_sum(msg, dst, num_segments=N)
        out, h = prop(agg, conv_b.reshape(1, D), conv_W, dinv2)

    batch2 = batch.reshape(N, 1)
    wia = lstm_Wih[:, :D].T
    wib = lstm_Wih[:, D:].T
    whh = lstm_Whh.T
    bsum = (lstm_bih + lstm_bhh).reshape(1, 4 * D)

    lstm = pl.pallas_call(
        _lstm_body,
        in_specs=[pl.BlockSpec((B, D), lambda: (0, 0)),
                  pl.BlockSpec((B, D), lambda: (0, 0)),
                  pl.BlockSpec((B, 1), lambda: (0, 0)),
                  pl.BlockSpec((B, D), lambda: (0, 0)),
                  pl.BlockSpec((B, D), lambda: (0, 0)),
                  pl.BlockSpec((D, 4 * D), lambda: (0, 0)),
                  pl.BlockSpec((D, 4 * D), lambda: (0, 0)),
                  pl.BlockSpec((D, 4 * D), lambda: (0, 0)),
                  pl.BlockSpec((1, 4 * D), lambda: (0, 0))],
        out_specs=(pl.BlockSpec((B, D), lambda: (0, 0)),
                   pl.BlockSpec((B, D), lambda: (0, 0))),
        out_shape=(jax.ShapeDtypeStruct((B, D), f32),
                   jax.ShapeDtypeStruct((B, D), f32)),
    )

    emax_call = pl.pallas_call(
        _emax_body,
        grid=grid,
        in_specs=[_node_spec(D), _node_spec(1), _full_spec((B, D))],
        out_specs=_full_spec((1, B)),
        out_shape=jax.ShapeDtypeStruct((1, B), f32),
    )

    att_call = pl.pallas_call(
        _att_body,
        grid=grid,
        in_specs=[_node_spec(D), _node_spec(1), _full_spec((B, D)),
                  _full_spec((1, B))],
        out_specs=(_full_spec((B, 1)), _full_spec((B, D))),
        out_shape=(jax.ShapeDtypeStruct((B, 1), f32),
                   jax.ShapeDtypeStruct((B, D), f32)),
    )

    q = jnp.zeros((B, D), f32)
    hs = jnp.zeros((B, D), f32)
    cs = jnp.zeros((B, D), f32)
    rnum = jnp.zeros((B, D), f32)
    denom = jnp.ones((B, 1), f32)

    for _ in range(6):
        hs, cs = lstm(q, rnum, denom, hs, cs, wia, wib, whh, bsum)
        q = hs
        emax = emax_call(out, batch2, q)
        denom, rnum = att_call(out, batch2, q, emax)

    y = pl.pallas_call(
        _head_body,
        in_specs=[pl.BlockSpec((B, D), lambda: (0, 0)),
                  pl.BlockSpec((B, D), lambda: (0, 0)),
                  pl.BlockSpec((B, 1), lambda: (0, 0)),
                  pl.BlockSpec((D, D), lambda: (0, 0)),
                  pl.BlockSpec((D, D), lambda: (0, 0)),
                  pl.BlockSpec((1, D), lambda: (0, 0)),
                  pl.BlockSpec((D, 12), lambda: (0, 0)),
                  pl.BlockSpec((1, 12), lambda: (0, 0))],
        out_specs=pl.BlockSpec((B, 12), lambda: (0, 0)),
        out_shape=jax.ShapeDtypeStruct((B, 12), f32),
    )(q, rnum, denom, lin1_W[:D], lin1_W[D:], lin1_b.reshape(1, D),
      lin2_W, lin2_b.reshape(1, 12))

    return y


def kernel(x, edge_index, batch, lin0_W, lin0_b, conv_W, conv_b,
           lstm_Wih, lstm_Whh, lstm_bih, lstm_bhh, lin1_W, lin1_b,
           lin2_W, lin2_b):
    return _impl(x, edge_index, batch, lin0_W, lin0_b, conv_W, conv_b,
                 lstm_Wih, lstm_Whh, lstm_bih, lstm_bhh, lin1_W, lin1_b,
                 lin2_W, lin2_b)
